# Initial kernel scaffold; baseline (speedup 1.0000x reference)
import jax
import jax.numpy as jnp
from jax.experimental import pallas as pl


def kernel(x, edge_index, W_ego, b_ego, W_n1, b_n1, W_n2, b_n2, W_comb, b_comb, W_out, b_out):
    raise NotImplementedError("write your pallas kernel here")


# speedup vs baseline: 5.2892x; 5.2892x over previous
import functools

import jax
import jax.numpy as jnp
from jax import lax
from jax.experimental import pallas as pl
from jax.experimental.pallas import tpu as pltpu
from jax.experimental.pallas import tpu_sc as plsc

N = 10000
E = 320000
D = 128
H = 128
O = 64

NC = 2
NS = 16
NW = NC * NS
NPAD = 10240
W_AUG = D + 16
EPW = E // NW
CHUNK = 80
NCHUNK = EPW // CHUNK
ROWS_PER_SUB = NPAD // NS


def _aggregate_sc(tab, rowi, coli, zeros):
    @functools.partial(
        pl.kernel,
        mesh=plsc.VectorSubcoreMesh(core_axis_name="c", subcore_axis_name="s"),
        compiler_params=pltpu.CompilerParams(use_tc_tiling_on_sc=False),
        out_type=jax.ShapeDtypeStruct((NC, NPAD, W_AUG), jnp.float32),
        scratch_types=[
            pltpu.VMEM((NCHUNK, CHUNK), jnp.int32),
            pltpu.VMEM((NCHUNK, CHUNK), jnp.int32),
            pltpu.VMEM((CHUNK, W_AUG), jnp.float32),
            pltpu.VMEM_SHARED((NPAD, W_AUG), jnp.float32),
        ],
    )
    def agg(tab_hbm, rowi_hbm, coli_hbm, zeros_hbm, out_hbm,
            rowbuf, colbuf, rows_v, acc_sh):
        c = lax.axis_index("c")
        s = lax.axis_index("s")
        wid = c * NS + s
        sl = pl.ds(s * ROWS_PER_SUB, ROWS_PER_SUB)
        pltpu.sync_copy(zeros_hbm.at[sl], acc_sh.at[sl])
        pltpu.sync_copy(rowi_hbm.at[wid], rowbuf)
        pltpu.sync_copy(coli_hbm.at[wid], colbuf)
        plsc.subcore_barrier()

        def body(k, carry):
            pltpu.sync_copy(tab_hbm.at[rowbuf.at[k]], rows_v)
            pltpu.sync_copy(rows_v, acc_sh.at[colbuf.at[k]], add=True)
            return carry

        lax.fori_loop(0, NCHUNK, body, 0)
        plsc.subcore_barrier()
        pltpu.sync_copy(acc_sh.at[sl], out_hbm.at[c].at[sl])

    return agg(tab, rowi, coli, zeros)


def _combine_tc(acc):
    BLK = 512

    def body(acc_ref, out_ref):
        a = acc_ref[0] + acc_ref[1]
        deg = a[:, D:D + 1]
        inv = 1.0 / jnp.maximum(deg, 1.0)
        cols = lax.broadcasted_iota(jnp.int32, (BLK, W_AUG), 1)
        out_ref[...] = jnp.where(
            cols < D, a * inv, jnp.where(cols == D, 1.0, 0.0))

    return pl.pallas_call(
        body,
        grid=(NPAD // BLK,),
        in_specs=[pl.BlockSpec((NC, BLK, W_AUG), lambda i: (0, i, 0))],
        out_specs=pl.BlockSpec((BLK, W_AUG), lambda i: (i, 0)),
        out_shape=jax.ShapeDtypeStruct((NPAD, W_AUG), jnp.float32),
    )(acc)


def _final_tc(x, n1aug, acc2, W_ego, b_ego, W_n1, b_n1, W_n2, b_n2,
              W_comb, b_comb, W_out, b_out):
    BLK = 400

    def body(x_ref, n1_ref, acc_ref, we_ref, be_ref, w1_ref, b1_ref,
             w2_ref, b2_ref, wc_ref, bc_ref, wo_ref, bo_ref, out_ref):
        a = acc_ref[0] + acc_ref[1]
        inv = 1.0 / jnp.maximum(a[:, D:D + 1], 1.0)
        n2 = a[:, :D] * inv
        n1 = n1_ref[:, :D]
        f32 = jnp.float32
        h_ego = jnp.dot(x_ref[...], we_ref[...], preferred_element_type=f32) + be_ref[...]
        h_n1 = jnp.dot(n1, w1_ref[...], preferred_element_type=f32) + b1_ref[...]
        h_n2 = jnp.dot(n2, w2_ref[...], preferred_element_type=f32) + b2_ref[...]
        h = (jnp.dot(h_ego, wc_ref[:D], preferred_element_type=f32)
             + jnp.dot(h_n1, wc_ref[D:2 * D], preferred_element_type=f32)
             + jnp.dot(h_n2, wc_ref[2 * D:], preferred_element_type=f32)
             + bc_ref[...])
        h = jnp.maximum(h, 0.0)
        out_ref[...] = jnp.dot(h, wo_ref[...], preferred_element_type=f32) + bo_ref[...]

    full = lambda shape: pl.BlockSpec(shape, lambda i: tuple(0 for _ in shape))
    return pl.pallas_call(
        body,
        grid=(N // BLK,),
        in_specs=[
            pl.BlockSpec((BLK, D), lambda i: (i, 0)),
            pl.BlockSpec((BLK, W_AUG), lambda i: (i, 0)),
            pl.BlockSpec((NC, BLK, W_AUG), lambda i: (0, i, 0)),
            full((D, H)), full((1, H)),
            full((D, H)), full((1, H)),
            full((D, H)), full((1, H)),
            full((3 * H, H)), full((1, H)),
            full((H, O)), full((1, O)),
        ],
        out_specs=pl.BlockSpec((BLK, O), lambda i: (i, 0)),
        out_shape=jax.ShapeDtypeStruct((N, O), jnp.float32),
    )(x, n1aug, acc2, W_ego, b_ego.reshape(1, H), W_n1, b_n1.reshape(1, H),
      W_n2, b_n2.reshape(1, H), W_comb, b_comb.reshape(1, H),
      W_out, b_out.reshape(1, O))


def kernel(x, edge_index, W_ego, b_ego, W_n1, b_n1, W_n2, b_n2,
           W_comb, b_comb, W_out, b_out):
    rowi = edge_index[0].reshape(NW, NCHUNK, CHUNK)
    coli = edge_index[1].reshape(NW, NCHUNK, CHUNK)
    x_aug = jnp.concatenate(
        [x, jnp.ones((N, 1), jnp.float32),
         jnp.zeros((N, W_AUG - D - 1), jnp.float32)], axis=1)
    x_aug = jnp.pad(x_aug, ((0, NPAD - N), (0, 0)))
    zeros = jnp.zeros((NPAD, W_AUG), jnp.float32)

    acc1 = _aggregate_sc(x_aug, rowi, coli, zeros)
    n1_aug = _combine_tc(acc1)
    acc2 = _aggregate_sc(n1_aug, rowi, coli, zeros)
    return _final_tc(x, n1_aug, acc2, W_ego, b_ego, W_n1, b_n1,
                     W_n2, b_n2, W_comb, b_comb, W_out, b_out)


# speedup vs baseline: 7.7837x; 1.4716x over previous
import functools

import jax
import jax.numpy as jnp
from jax import lax
from jax.experimental import pallas as pl
from jax.experimental.pallas import tpu as pltpu
from jax.experimental.pallas import tpu_sc as plsc

N = 10000
E = 320000
D = 128
H = 128
O = 64

NC = 2
NS = 16
NW = NC * NS
NPAD = 10240
W_AUG = D + 16
EPW = E // NW
CHUNK = 100
NCHUNK = EPW // CHUNK
ROWS_PER_SUB = NPAD // NS


def _aggregate_sc(tab, edges, zeros):
    @functools.partial(
        pl.kernel,
        mesh=plsc.VectorSubcoreMesh(core_axis_name="c", subcore_axis_name="s"),
        compiler_params=pltpu.CompilerParams(use_tc_tiling_on_sc=False),
        out_type=jax.ShapeDtypeStruct((NC, NPAD, W_AUG), jnp.float32),
        scratch_types=[
            pltpu.VMEM((4, 2, CHUNK), jnp.int32),
            pltpu.VMEM((CHUNK, W_AUG), jnp.float32),
            pltpu.VMEM((CHUNK, W_AUG), jnp.float32),
            pltpu.VMEM_SHARED((NPAD, W_AUG), jnp.float32),
            pltpu.SemaphoreType.DMA,
            pltpu.SemaphoreType.DMA,
            pltpu.SemaphoreType.DMA,
            pltpu.SemaphoreType.DMA,
            pltpu.SemaphoreType.DMA,
            pltpu.SemaphoreType.DMA,
        ],
    )
    def agg(tab_hbm, edges_hbm, zeros_hbm, out_hbm,
            idxbuf, buf0, buf1, acc_sh, is0, is1, is2, is3, gs0, gs1):
        c = lax.axis_index("c")
        s = lax.axis_index("s")
        wid = c * NS + s
        isem = (is0, is1, is2, is3)
        gsem = (gs0, gs1)
        dbuf = (buf0, buf1)
        sl = pl.ds(s * ROWS_PER_SUB, ROWS_PER_SUB)
        pltpu.sync_copy(zeros_hbm.at[sl], acc_sh.at[sl])
        plsc.subcore_barrier()

        def idx_issue(k, slot):
            pltpu.async_copy(edges_hbm.at[wid].at[k], idxbuf.at[slot],
                             isem[slot])

        def idx_wait(slot):
            pltpu.make_async_copy(edges_hbm.at[0].at[0], idxbuf.at[slot],
                                  isem[slot]).wait()

        def g_issue(slot, b):
            pltpu.async_copy(tab_hbm.at[idxbuf.at[slot, 0]], dbuf[b], gsem[b])

        def g_wait(b):
            pltpu.make_async_copy(tab_hbm.at[idxbuf.at[0, 0]], dbuf[b],
                                  gsem[b]).wait()

        def scatter(slot, b):
            pltpu.sync_copy(dbuf[b], acc_sh.at[idxbuf.at[slot, 1]], add=True)

        for j in range(4):
            idx_issue(j, j)
        idx_wait(0)
        g_issue(0, 0)
        idx_wait(1)
        g_issue(1, 1)

        def body(g, carry):
            for j in range(4):
                k = 4 * g + j
                b = j % 2
                g_wait(b)
                scatter(j, b)

                @pl.when(k + 4 < NCHUNK)
                def _():
                    idx_issue(k + 4, j)

                @pl.when(k + 2 < NCHUNK)
                def _():
                    idx_wait((j + 2) % 4)
                    g_issue((j + 2) % 4, b)

            return carry

        lax.fori_loop(0, NCHUNK // 4, body, 0)
        plsc.subcore_barrier()
        pltpu.sync_copy(acc_sh.at[sl], out_hbm.at[c].at[sl])

    return agg(tab, edges, zeros)

    return agg(tab, rowi, coli, zeros)


def _combine_tc(acc):
    BLK = 512

    def body(acc_ref, out_ref):
        a = acc_ref[0] + acc_ref[1]
        deg = a[:, D:D + 1]
        inv = 1.0 / jnp.maximum(deg, 1.0)
        cols = lax.broadcasted_iota(jnp.int32, (BLK, W_AUG), 1)
        out_ref[...] = jnp.where(
            cols < D, a * inv, jnp.where(cols == D, 1.0, 0.0))

    return pl.pallas_call(
        body,
        grid=(NPAD // BLK,),
        in_specs=[pl.BlockSpec((NC, BLK, W_AUG), lambda i: (0, i, 0))],
        out_specs=pl.BlockSpec((BLK, W_AUG), lambda i: (i, 0)),
        out_shape=jax.ShapeDtypeStruct((NPAD, W_AUG), jnp.float32),
    )(acc)


def _final_tc(x, n1aug, acc2, W_ego, b_ego, W_n1, b_n1, W_n2, b_n2,
              W_comb, b_comb, W_out, b_out):
    BLK = 400

    def body(x_ref, n1_ref, acc_ref, we_ref, be_ref, w1_ref, b1_ref,
             w2_ref, b2_ref, wc_ref, bc_ref, wo_ref, bo_ref, out_ref):
        a = acc_ref[0] + acc_ref[1]
        inv = 1.0 / jnp.maximum(a[:, D:D + 1], 1.0)
        n2 = a[:, :D] * inv
        n1 = n1_ref[:, :D]
        f32 = jnp.float32
        h_ego = jnp.dot(x_ref[...], we_ref[...], preferred_element_type=f32) + be_ref[...]
        h_n1 = jnp.dot(n1, w1_ref[...], preferred_element_type=f32) + b1_ref[...]
        h_n2 = jnp.dot(n2, w2_ref[...], preferred_element_type=f32) + b2_ref[...]
        h = (jnp.dot(h_ego, wc_ref[:D], preferred_element_type=f32)
             + jnp.dot(h_n1, wc_ref[D:2 * D], preferred_element_type=f32)
             + jnp.dot(h_n2, wc_ref[2 * D:], preferred_element_type=f32)
             + bc_ref[...])
        h = jnp.maximum(h, 0.0)
        out_ref[...] = jnp.dot(h, wo_ref[...], preferred_element_type=f32) + bo_ref[...]

    full = lambda shape: pl.BlockSpec(shape, lambda i: tuple(0 for _ in shape))
    return pl.pallas_call(
        body,
        grid=(N // BLK,),
        in_specs=[
            pl.BlockSpec((BLK, D), lambda i: (i, 0)),
            pl.BlockSpec((BLK, W_AUG), lambda i: (i, 0)),
            pl.BlockSpec((NC, BLK, W_AUG), lambda i: (0, i, 0)),
            full((D, H)), full((1, H)),
            full((D, H)), full((1, H)),
            full((D, H)), full((1, H)),
            full((3 * H, H)), full((1, H)),
            full((H, O)), full((1, O)),
        ],
        out_specs=pl.BlockSpec((BLK, O), lambda i: (i, 0)),
        out_shape=jax.ShapeDtypeStruct((N, O), jnp.float32),
    )(x, n1aug, acc2, W_ego, b_ego.reshape(1, H), W_n1, b_n1.reshape(1, H),
      W_n2, b_n2.reshape(1, H), W_comb, b_comb.reshape(1, H),
      W_out, b_out.reshape(1, O))


def kernel(x, edge_index, W_ego, b_ego, W_n1, b_n1, W_n2, b_n2,
           W_comb, b_comb, W_out, b_out):
    edges = edge_index.reshape(2, NW, NCHUNK, CHUNK).transpose(1, 2, 0, 3)
    x_aug = jnp.concatenate(
        [x, jnp.ones((N, 1), jnp.float32),
         jnp.zeros((N, W_AUG - D - 1), jnp.float32)], axis=1)
    x_aug = jnp.pad(x_aug, ((0, NPAD - N), (0, 0)))
    zeros = jnp.zeros((NPAD, W_AUG), jnp.float32)

    acc1 = _aggregate_sc(x_aug, edges, zeros)
    n1_aug = _combine_tc(acc1)
    acc2 = _aggregate_sc(n1_aug, edges, zeros)
    return _final_tc(x, n1_aug, acc2, W_ego, b_ego, W_n1, b_n1,
                     W_n2, b_n2, W_comb, b_comb, W_out, b_out)


# speedup vs baseline: 10.4658x; 1.3446x over previous
import functools

import jax
import jax.numpy as jnp
from jax import lax
from jax.experimental import pallas as pl
from jax.experimental.pallas import tpu as pltpu
from jax.experimental.pallas import tpu_sc as plsc

N = 10000
E = 320000
D = 128
H = 128
O = 64

NC = 2
NS = 16
NW = NC * NS
EPW = E // NW
CHUNK = 80
NCHUNK = EPW // CHUNK
NBUF = 4
NSLOT = 8
ROWS_PER_SUB = N // NS


def _aggregate_sc(tab, edges, zeros2, zeros1, ones, with_deg):
    if with_deg:
        out_type = (jax.ShapeDtypeStruct((NC, N, D), jnp.float32),
                    jax.ShapeDtypeStruct((NC, N), jnp.float32))
    else:
        out_type = jax.ShapeDtypeStruct((NC, N, D), jnp.float32)

    @functools.partial(
        pl.kernel,
        mesh=plsc.VectorSubcoreMesh(core_axis_name="c", subcore_axis_name="s"),
        compiler_params=pltpu.CompilerParams(use_tc_tiling_on_sc=False),
        out_type=out_type,
        scratch_types=[
            pltpu.VMEM((NSLOT, 2, CHUNK), jnp.int32),
            pltpu.VMEM((CHUNK, D), jnp.float32),
            pltpu.VMEM((CHUNK, D), jnp.float32),
            pltpu.VMEM((CHUNK, D), jnp.float32),
            pltpu.VMEM((CHUNK, D), jnp.float32),
            pltpu.VMEM((CHUNK,), jnp.float32),
            pltpu.VMEM_SHARED((N, D), jnp.float32),
            pltpu.VMEM_SHARED((N,), jnp.float32),
        ] + [pltpu.SemaphoreType.DMA] * (NSLOT + 2 * NBUF),
    )
    def agg(tab_hbm, edges_hbm, z2_hbm, z1_hbm, ones_hbm, *rest):
        if with_deg:
            out_hbm, outdeg_hbm = rest[0], rest[1]
            rest = rest[2:]
        else:
            out_hbm = rest[0]
            rest = rest[1:]
        idxbuf, b0, b1, b2, b3, onesbuf, acc_sh, deg_sh = rest[:8]
        sems = rest[8:]
        isem = sems[:NSLOT]
        gsem = sems[NSLOT:NSLOT + NBUF]
        ssem = sems[NSLOT + NBUF:]
        dbuf = (b0, b1, b2, b3)

        c = lax.axis_index("c")
        s = lax.axis_index("s")
        wid = c * NS + s
        sl = pl.ds(s * ROWS_PER_SUB, ROWS_PER_SUB)

        def idx_issue(k, slot):
            pltpu.async_copy(edges_hbm.at[0].at[wid].at[k],
                             idxbuf.at[slot, 0], isem[slot])
            pltpu.async_copy(edges_hbm.at[1].at[wid].at[k],
                             idxbuf.at[slot, 1], isem[slot])

        def idx_wait(slot):
            for half in (0, 1):
                pltpu.make_async_copy(edges_hbm.at[0].at[0].at[0],
                                      idxbuf.at[slot, half],
                                      isem[slot]).wait()

        def g_issue(k_slot, b):
            pltpu.async_copy(tab_hbm.at[idxbuf.at[k_slot, 0]], dbuf[b],
                             gsem[b])

        def g_wait(b):
            pltpu.make_async_copy(tab_hbm.at[idxbuf.at[0, 0]], dbuf[b],
                                  gsem[b]).wait()

        def s_issue(slot, b):
            pltpu.async_copy(dbuf[b], acc_sh.at[idxbuf.at[slot, 1]],
                             ssem[b], add=True)
            if with_deg:
                pltpu.async_copy(onesbuf, deg_sh.at[idxbuf.at[slot, 1]],
                                 ssem[b], add=True)

        def s_wait(b):
            pltpu.make_async_copy(z2_hbm.at[pl.ds(0, CHUNK)], dbuf[b],
                                  ssem[b]).wait()
            if with_deg:
                pltpu.make_async_copy(z1_hbm.at[pl.ds(0, CHUNK)], onesbuf,
                                      ssem[b]).wait()

        def step(k, j4, j8, do_c, do_d, do_ef):
            g_wait(j4)
            s_issue(j8, j4)
            if do_c:
                s_wait((j4 + 2) % NBUF)
            if do_d:
                idx_issue(k + 6, (j8 + 6) % NSLOT)
            if do_ef:
                idx_wait((j8 + 2) % NSLOT)
                g_issue((j8 + 2) % NSLOT, (j4 + 2) % NBUF)

        pltpu.sync_copy(z2_hbm.at[sl], acc_sh.at[sl])
        if with_deg:
            pltpu.sync_copy(ones_hbm, onesbuf)

            @pl.when(s == 0)
            def _():
                pltpu.sync_copy(z1_hbm, deg_sh)

        for m in range(6):
            idx_issue(m, m)
        idx_wait(0)
        g_issue(0, 0)
        idx_wait(1)
        g_issue(1, 1)
        plsc.subcore_barrier()

        for k in range(8):
            step(k, k % NBUF, k % NSLOT, k >= 2, True, True)

        def body(g, carry):
            for j in range(8):
                step(8 * g + j, j % NBUF, j, True, True, True)
            return carry

        gb = (NCHUNK - 14) // 8 + 1
        lax.fori_loop(1, gb, body, 0)

        for k in range(8 * gb, NCHUNK):
            step(k, k % NBUF, k % NSLOT, True, k + 6 < NCHUNK,
                 k + 2 < NCHUNK)
        s_wait((NCHUNK - 2) % NBUF)
        s_wait((NCHUNK - 1) % NBUF)

        plsc.subcore_barrier()
        pltpu.sync_copy(acc_sh.at[sl], out_hbm.at[c].at[sl])
        if with_deg:

            @pl.when(s == 0)
            def _():
                pltpu.sync_copy(deg_sh, outdeg_hbm.at[c])

    return agg(tab, edges, zeros2, zeros1, ones)


def _combine_tc(acc, deg_r):
    BLK = 1000

    def body(acc_ref, deg_ref, out_ref):
        a = acc_ref[0] + acc_ref[1]
        dg = deg_ref[0] + deg_ref[1]
        out_ref[...] = a * (1.0 / jnp.maximum(dg, 1.0))

    return pl.pallas_call(
        body,
        grid=(N // BLK,),
        in_specs=[pl.BlockSpec((NC, BLK, D), lambda i: (0, i, 0)),
                  pl.BlockSpec((NC, BLK, 1), lambda i: (0, i, 0))],
        out_specs=pl.BlockSpec((BLK, D), lambda i: (i, 0)),
        out_shape=jax.ShapeDtypeStruct((N, D), jnp.float32),
    )(acc, deg_r)


def _final_tc(x, n1, acc2, deg_r, W_ego, b_ego, W_n1, b_n1, W_n2, b_n2,
              W_comb, b_comb, W_out, b_out):
    BLK = 1000

    def body(x_ref, n1_ref, acc_ref, deg_ref, we_ref, be_ref, w1_ref, b1_ref,
             w2_ref, b2_ref, wc_ref, bc_ref, wo_ref, bo_ref, out_ref):
        a = acc_ref[0] + acc_ref[1]
        dg = deg_ref[0] + deg_ref[1]
        n2 = a * (1.0 / jnp.maximum(dg, 1.0))
        f32 = jnp.float32
        h_ego = jnp.dot(x_ref[...], we_ref[...],
                        preferred_element_type=f32) + be_ref[...]
        h_n1 = jnp.dot(n1_ref[...], w1_ref[...],
                       preferred_element_type=f32) + b1_ref[...]
        h_n2 = jnp.dot(n2, w2_ref[...], preferred_element_type=f32) + b2_ref[...]
        h = (jnp.dot(h_ego, wc_ref[:D], preferred_element_type=f32)
             + jnp.dot(h_n1, wc_ref[D:2 * D], preferred_element_type=f32)
             + jnp.dot(h_n2, wc_ref[2 * D:], preferred_element_type=f32)
             + bc_ref[...])
        h = jnp.maximum(h, 0.0)
        out_ref[...] = jnp.dot(h, wo_ref[...],
                               preferred_element_type=f32) + bo_ref[...]

    full = lambda shape: pl.BlockSpec(shape, lambda i: tuple(0 for _ in shape))
    return pl.pallas_call(
        body,
        grid=(N // BLK,),
        in_specs=[
            pl.BlockSpec((BLK, D), lambda i: (i, 0)),
            pl.BlockSpec((BLK, D), lambda i: (i, 0)),
            pl.BlockSpec((NC, BLK, D), lambda i: (0, i, 0)),
            pl.BlockSpec((NC, BLK, 1), lambda i: (0, i, 0)),
            full((D, H)), full((1, H)),
            full((D, H)), full((1, H)),
            full((D, H)), full((1, H)),
            full((3 * H, H)), full((1, H)),
            full((H, O)), full((1, O)),
        ],
        out_specs=pl.BlockSpec((BLK, O), lambda i: (i, 0)),
        out_shape=jax.ShapeDtypeStruct((N, O), jnp.float32),
    )(x, n1, acc2, deg_r, W_ego, b_ego.reshape(1, H), W_n1,
      b_n1.reshape(1, H), W_n2, b_n2.reshape(1, H), W_comb,
      b_comb.reshape(1, H), W_out, b_out.reshape(1, O))


def kernel(x, edge_index, W_ego, b_ego, W_n1, b_n1, W_n2, b_n2,
           W_comb, b_comb, W_out, b_out):
    edges = edge_index.reshape(2, NW, NCHUNK, CHUNK)
    zeros2 = jnp.zeros((N, D), jnp.float32)
    zeros1 = jnp.zeros((N,), jnp.float32)
    ones = jnp.ones((CHUNK,), jnp.float32)

    acc1, deg = _aggregate_sc(x, edges, zeros2, zeros1, ones, with_deg=True)
    deg_r = deg.reshape(NC, N, 1)
    n1 = _combine_tc(acc1, deg_r)
    acc2 = _aggregate_sc(n1, edges, zeros2, zeros1, ones, with_deg=False)
    return _final_tc(x, n1, acc2, deg_r, W_ego, b_ego, W_n1, b_n1,
                     W_n2, b_n2, W_comb, b_comb, W_out, b_out)
